# Initial kernel scaffold; baseline (speedup 1.0000x reference)
import jax
import jax.numpy as jnp
from jax.experimental import pallas as pl


def kernel(x, We0, be0, We1, be1, Wi, bi, Wd1, bd1, Wc1, bc1, Wd2, bd2, Wc2, bc2, Wd3, bd3, Wc3, bc3, Wo, bo):
    raise NotImplementedError("write your pallas kernel here")


# speedup vs baseline: 1.0031x; 1.0031x over previous
import jax
import jax.numpy as jnp
from jax.experimental import pallas as pl
from jax.experimental.pallas import tpu as pltpu

_DN = ('NCHW', 'OIHW', 'NCHW')
_NBINS = 256
_NPIX = 256
_NPATCH = 256


def _conv(x, w, b, stride=1, pad=1):
    y = jax.lax.conv_general_dilated(x, w, (stride, stride), [(pad, pad), (pad, pad)],
                                     dimension_numbers=_DN)
    return y + b[None, :, None, None]


def _deconv(x, w, b):
    wt = jnp.flip(w, (2, 3)).transpose(1, 0, 2, 3)
    y = jax.lax.conv_general_dilated(x, wt, (1, 1), [(1, 2), (1, 2)],
                                     lhs_dilation=(2, 2), dimension_numbers=_DN)
    return y + b[None, :, None, None]


def _ent_kernel(v_ref, ent_ref, pdf_scr):
    v = v_ref[0]

    def body(k, carry):
        rows = []
        for d in range(8):
            m = k * 8 + d
            b = m.astype(jnp.float32) * (1.0 / 255.0)
            t = (v - b) * 100.0
            w = jnp.exp(-0.5 * (t * t))
            rows.append(jnp.sum(w, axis=0, keepdims=True))
        pdf_scr[pl.ds(k, 1)] = jnp.concatenate(rows, axis=0)[None]
        return carry

    jax.lax.fori_loop(0, _NBINS // 8, body, 0)
    pdf = pdf_scr[...].reshape(_NBINS, _NPATCH)
    s = jnp.sum(pdf, axis=0, keepdims=True)
    pn = jnp.maximum(pdf * (1.0 / s), 1e-10)
    ent_ref[...] = -jnp.sum(pn * jnp.log2(pn), axis=0, keepdims=True)[None]


def _entropy_map(gray):
    bsz = gray.shape[0]
    p = gray.reshape(bsz, 16, 16, 16, 16).transpose(0, 2, 4, 1, 3)
    p = p.reshape(bsz, _NPIX, _NPATCH)
    ent = pl.pallas_call(
        _ent_kernel,
        grid=(bsz,),
        in_specs=[pl.BlockSpec((1, _NPIX, _NPATCH), lambda b: (b, 0, 0))],
        out_specs=pl.BlockSpec((1, 1, _NPATCH), lambda b: (b, 0, 0)),
        out_shape=jax.ShapeDtypeStruct((bsz, 1, _NPATCH), jnp.float32),
        scratch_shapes=[pltpu.VMEM((_NBINS // 8, 8, _NPATCH), jnp.float32)],
        compiler_params=pltpu.CompilerParams(
            dimension_semantics=("parallel",)),
    )(p)
    return ent.reshape(bsz, 16, 16)


def kernel(x, We0, be0, We1, be1, Wi, bi, Wd1, bd1, Wc1, bc1, Wd2, bd2,
           Wc2, bc2, Wd3, bd3, Wc3, bc3, Wo, bo):
    lat_fine = _conv(x, We0, be0, stride=8)
    lat_coarse = _conv(x, We1, be1, stride=16)
    gray = 0.299 * x[:, 0] + 0.587 * x[:, 1] + 0.114 * x[:, 2]
    ent = _entropy_map(gray)
    thr = jnp.quantile(ent.reshape(-1), 0.5)
    grain = (ent > thr).astype(x.dtype)
    coarse_up = jnp.repeat(jnp.repeat(lat_coarse, 2, axis=2), 2, axis=3)
    g = jnp.repeat(jnp.repeat(grain, 2, axis=1), 2, axis=2)[:, None]
    routed = g * lat_fine + (1.0 - g) * coarse_up
    h = _conv(routed, Wi, bi)
    h = jax.nn.relu(_conv(jax.nn.relu(_deconv(h, Wd1, bd1)), Wc1, bc1))
    h = jax.nn.relu(_conv(jax.nn.relu(_deconv(h, Wd2, bd2)), Wc2, bc2))
    h = jax.nn.relu(_conv(jax.nn.relu(_deconv(h, Wd3, bd3)), Wc3, bc3))
    rec = jnp.tanh(_conv(h, Wo, bo))
    return rec, routed, grain, ent
